# Initial kernel scaffold; baseline (speedup 1.0000x reference)
import jax
import jax.numpy as jnp
from jax.experimental import pallas as pl


def kernel(x, tau):
    raise NotImplementedError("write your pallas kernel here")


# speedup vs baseline: 83.6077x; 83.6077x over previous
import functools

import jax
import jax.numpy as jnp
from jax.experimental import pallas as pl

_TOPK = 0.1
_KH, _KW = 5, 5
_INT_MIN = -2147483648
_INT_MAX = 2147483647


def _sortable_key(f):
    b = jax.lax.bitcast_convert_type(f, jnp.int32)
    return jnp.where(b < 0, b ^ 0x7FFFFFFF, b)


def _mean_body(x_ref, t_ref):
    t_ref[0] = jnp.mean(x_ref[0], axis=0)


def _select_body(t_ref, tau_ref, w_ref, *, hout, wout, kkeep):
    T = t_ref[0]
    H, W = T.shape
    key = _sortable_key(T)

    p = jax.lax.broadcasted_iota(jnp.int32, (H, 1), 0)
    q = jax.lax.broadcasted_iota(jnp.int32, (1, W), 1)
    rms = [jnp.logical_and(p >= i, p < i + hout) for i in range(_KH)]
    cms = [jnp.logical_and(q >= j, q < j + wout) for j in range(_KW)]
    wms = [jnp.logical_and(rms[i], cms[j]) for i in range(_KH) for j in range(_KW)]

    nwin = _KH * _KW

    def body(_, carry):
        los, his = carry
        nlos, nhis = [], []
        for w in range(nwin):
            lo, hi = los[w], his[w]
            x = lo ^ hi
            mid = (lo & hi) + (x >> 1) + (x & 1)
            cnt = jnp.sum(
                jnp.logical_and(key >= mid, wms[w]).astype(jnp.int32))
            pred = cnt >= kkeep
            nlos.append(jnp.where(pred, mid, lo))
            nhis.append(jnp.where(pred, hi, mid - 1))
        return tuple(nlos), tuple(nhis)

    init = (tuple(jnp.int32(_INT_MIN) for _ in range(nwin)),
            tuple(jnp.int32(_INT_MAX) for _ in range(nwin)))
    los, _ = jax.lax.fori_loop(0, 32, body, init)

    big = jnp.int32(_INT_MAX)
    thr = None
    for j in range(_KW):
        rmin = None
        for i in range(_KH):
            v = jnp.where(rms[i], los[i * _KW + j], big)
            rmin = v if rmin is None else jnp.minimum(rmin, v)
        v = jnp.where(cms[j], rmin, big)
        thr = v if thr is None else jnp.minimum(thr, v)

    mask = (key >= thr).astype(jnp.float32)
    tau = tau_ref[0, 0]
    w_ref[0] = mask * tau + (1.0 - tau)


def _apply_body(x_ref, w_ref, o_ref):
    o_ref[0] = x_ref[0] * w_ref[0][None]


@jax.jit
def kernel(x, tau):
    n, c, h, w = x.shape
    hout, wout = h - _KH + 1, w - _KW + 1
    kkeep = max(int(_TOPK * (hout * wout)), 1)

    ht = 32 if h % 32 == 0 else h
    nh = h // ht

    tmean = pl.pallas_call(
        _mean_body,
        grid=(n, nh),
        in_specs=[pl.BlockSpec((1, c, ht, w), lambda b, i: (b, 0, i, 0))],
        out_specs=pl.BlockSpec((1, ht, w), lambda b, i: (b, i, 0)),
        out_shape=jax.ShapeDtypeStruct((n, h, w), jnp.float32),
    )(x)

    tau_arr = jnp.full((8, 128), tau, dtype=jnp.float32)
    wmap = pl.pallas_call(
        functools.partial(_select_body, hout=hout, wout=wout, kkeep=kkeep),
        grid=(n,),
        in_specs=[
            pl.BlockSpec((1, h, w), lambda b: (b, 0, 0)),
            pl.BlockSpec((8, 128), lambda b: (0, 0)),
        ],
        out_specs=pl.BlockSpec((1, h, w), lambda b: (b, 0, 0)),
        out_shape=jax.ShapeDtypeStruct((n, h, w), jnp.float32),
    )(tmean, tau_arr)

    out = pl.pallas_call(
        _apply_body,
        grid=(n, nh),
        in_specs=[
            pl.BlockSpec((1, c, ht, w), lambda b, i: (b, 0, i, 0)),
            pl.BlockSpec((1, ht, w), lambda b, i: (b, i, 0)),
        ],
        out_specs=pl.BlockSpec((1, c, ht, w), lambda b, i: (b, 0, i, 0)),
        out_shape=jax.ShapeDtypeStruct((n, c, h, w), jnp.float32),
    )(x, wmap)
    return out
